# baseline (speedup 1.0000x reference)
import jax
import jax.numpy as jnp
from jax.experimental import pallas as pl

_N = 2048
_K = 1024
_D = 64
_TILE = 2048


def _split_bf16(x):
    hi = x.astype(jnp.bfloat16)
    rem = x - hi.astype(jnp.float32)
    return hi, rem


def _split_bf16_trunc(x):
    hi_f = jax.lax.bitcast_convert_type(
        jax.lax.bitcast_convert_type(x, jnp.int32)
        & jnp.int32(0xFFFF0000 - 0x100000000),
        jnp.float32,
    )
    return hi_f.astype(jnp.bfloat16), x - hi_f


def _vq_body(ze_ref, w_ref, zq_ref, idx_ref, loss_ref):
    i = pl.program_id(0)
    ze = ze_ref[...]
    w = w_ref[...]
    wt = jnp.swapaxes(w, 0, 1)

    def _dot(a, b):
        return jax.lax.dot_general(
            a, b, (((1,), (0,)), ((), ())),
            preferred_element_type=jnp.float32,
        )

    ze_hi, ze_rem = _split_bf16(ze)
    ze_lo = ze_rem.astype(jnp.bfloat16)
    wt_hi, wt_rem = _split_bf16(wt)
    wt_lo = wt_rem.astype(jnp.bfloat16)
    s = _dot(ze_hi, wt_hi) + (_dot(ze_hi, wt_lo) + _dot(ze_lo, wt_hi))

    wn = jnp.sum(wt * wt, axis=0, keepdims=True)
    zn = jnp.sum(ze * ze, axis=1, keepdims=True)
    dist = (zn + wn) - 2.0 * s

    iota = jax.lax.broadcasted_iota(jnp.int32, dist.shape, 1)
    key = (jax.lax.bitcast_convert_type(dist, jnp.int32) & ~(_K - 1)) | iota
    k1 = jnp.min(key, axis=1, keepdims=True)
    m1 = key == k1
    key2 = jnp.where(m1, jnp.int32(0x7FFFFFFF), key)
    k2 = jnp.min(key2, axis=1, keepdims=True)
    m2 = key2 == k2
    i1 = k1 & (_K - 1)
    i2 = k2 & (_K - 1)

    oh = jnp.concatenate([m1, m2], axis=0).astype(jnp.bfloat16)
    w_hi, w_rem = _split_bf16_trunc(w)
    w_mid, w_rem2 = _split_bf16_trunc(w_rem)
    w_lo = w_rem2.astype(jnp.bfloat16)
    zqs = _dot(oh, w_hi) + (_dot(oh, w_mid) + _dot(oh, w_lo))
    zq1 = zqs[:_TILE]
    zq2 = zqs[_TILE:]

    d1 = jnp.sum((zq1 - ze) ** 2, axis=1, keepdims=True)
    d2 = jnp.sum((zq2 - ze) ** 2, axis=1, keepdims=True)
    use2 = (d2 < d1) | ((d2 == d1) & (i2 < i1))

    idx_ref[...] = jnp.where(use2, i2, i1)
    zq_ref[...] = jnp.where(use2, zq2, zq1)

    part = (jnp.sum(jnp.where(use2, d2, d1)) / (_N * _D)).reshape(1, 1)

    @pl.when(i == 0)
    def _():
        loss_ref[...] = part

    @pl.when(i > 0)
    def _():
        loss_ref[...] += part


def kernel(ze, embedW):
    n_tiles = _N // _TILE
    zq, idx, loss = pl.pallas_call(
        _vq_body,
        grid=(n_tiles,),
        in_specs=[
            pl.BlockSpec((_TILE, _D), lambda i: (i, 0)),
            pl.BlockSpec((_K, _D), lambda i: (0, 0)),
        ],
        out_specs=[
            pl.BlockSpec((_TILE, _D), lambda i: (i, 0)),
            pl.BlockSpec((_TILE, 1), lambda i: (i, 0)),
            pl.BlockSpec((1, 1), lambda i: (0, 0)),
        ],
        out_shape=[
            jax.ShapeDtypeStruct((_N, _D), jnp.float32),
            jax.ShapeDtypeStruct((_N, 1), jnp.int32),
            jax.ShapeDtypeStruct((1, 1), jnp.float32),
        ],
    )(ze, embedW)
    return (zq, loss.reshape(()), idx.reshape(-1))


# speedup vs baseline: 1.0196x; 1.0196x over previous
import jax
import jax.numpy as jnp
from jax.experimental import pallas as pl

_N = 2048
_K = 1024
_D = 64
_TILE = 2048


def _split_bf16(x):
    hi = x.astype(jnp.bfloat16)
    rem = x - hi.astype(jnp.float32)
    return hi, rem


def _split_bf16_trunc(x):
    hi_f = jax.lax.bitcast_convert_type(
        jax.lax.bitcast_convert_type(x, jnp.int32)
        & jnp.int32(0xFFFF0000 - 0x100000000),
        jnp.float32,
    )
    return hi_f.astype(jnp.bfloat16), x - hi_f


def _vq_body(ze_ref, w_ref, zq_ref, idx_ref, loss_ref):
    i = pl.program_id(0)
    ze = ze_ref[...]
    w = w_ref[...]
    wt = jnp.swapaxes(w, 0, 1)

    def _dot(a, b):
        return jax.lax.dot_general(
            a, b, (((1,), (0,)), ((), ())),
            preferred_element_type=jnp.float32,
        )

    ze_hi, ze_rem = _split_bf16(ze)
    ze_lo = ze_rem.astype(jnp.bfloat16)
    wt_hi, wt_rem = _split_bf16(wt)
    wt_lo = wt_rem.astype(jnp.bfloat16)
    s = _dot(ze_hi, wt_hi) + (_dot(ze_hi, wt_lo) + _dot(ze_lo, wt_hi))

    wn = jnp.sum(wt * wt, axis=0, keepdims=True)
    zn = jnp.sum(ze * ze, axis=1, keepdims=True)
    dist = jnp.maximum((zn + wn) - 2.0 * s, 0.0)

    iota = jax.lax.broadcasted_iota(jnp.int32, dist.shape, 1)
    key = (jax.lax.bitcast_convert_type(dist, jnp.int32) & ~(_K - 1)) | iota
    k1 = jnp.min(key, axis=1, keepdims=True)
    key2 = jnp.where(key == k1, jnp.int32(0x7FFFFFFF), key)
    k2 = jnp.min(key2, axis=1, keepdims=True)
    i1 = k1 & (_K - 1)
    i2 = k2 & (_K - 1)

    oh = jnp.concatenate(
        [(iota == i1).astype(jnp.bfloat16), (iota == i2).astype(jnp.bfloat16)],
        axis=0,
    )
    w_hi, w_rem = _split_bf16_trunc(w)
    w_mid, w_rem2 = _split_bf16_trunc(w_rem)
    w_lo = w_rem2.astype(jnp.bfloat16)
    zqs = _dot(oh, w_hi) + (_dot(oh, w_mid) + _dot(oh, w_lo))
    zq1 = zqs[:_TILE]
    zq2 = zqs[_TILE:]

    d1 = jnp.sum((zq1 - ze) ** 2, axis=1, keepdims=True)
    d2 = jnp.sum((zq2 - ze) ** 2, axis=1, keepdims=True)
    use2 = (d2 < d1) | ((d2 == d1) & (i2 < i1))

    idx_ref[...] = jnp.where(use2, i2, i1)
    zq_ref[...] = jnp.where(use2, zq2, zq1)

    part = (jnp.sum(jnp.where(use2, d2, d1)) / (_N * _D)).reshape(1, 1)

    @pl.when(i == 0)
    def _():
        loss_ref[...] = part

    @pl.when(i > 0)
    def _():
        loss_ref[...] += part


def kernel(ze, embedW):
    n_tiles = _N // _TILE
    zq, idx, loss = pl.pallas_call(
        _vq_body,
        grid=(n_tiles,),
        in_specs=[
            pl.BlockSpec((_TILE, _D), lambda i: (i, 0)),
            pl.BlockSpec((_K, _D), lambda i: (0, 0)),
        ],
        out_specs=[
            pl.BlockSpec((_TILE, _D), lambda i: (i, 0)),
            pl.BlockSpec((_TILE, 1), lambda i: (i, 0)),
            pl.BlockSpec((1, 1), lambda i: (0, 0)),
        ],
        out_shape=[
            jax.ShapeDtypeStruct((_N, _D), jnp.float32),
            jax.ShapeDtypeStruct((_N, 1), jnp.int32),
            jax.ShapeDtypeStruct((1, 1), jnp.float32),
        ],
    )(ze, embedW)
    return (zq, loss.reshape(()), idx.reshape(-1))
